# Initial kernel scaffold; baseline (speedup 1.0000x reference)
import jax
import jax.numpy as jnp
from jax.experimental import pallas as pl


def kernel(x, edge_index, train_mask, W, b):
    raise NotImplementedError("write your pallas kernel here")


# speedup vs baseline: 30.0030x; 30.0030x over previous
import functools

import jax
import jax.numpy as jnp
from jax import lax
from jax.experimental import pallas as pl
from jax.experimental.pallas import tpu as pltpu
from jax.experimental.pallas import tpu_sc as plsc

N = 10000
E = 320000
D = 128
DV = 144
NC = 2
NS = 16
NW = NC * NS
EPW = E // NW
CHUNK = 80
NCH = EPW // CHUNK
BN = 400
DEGW = 16

_mesh = plsc.VectorSubcoreMesh(core_axis_name="c", subcore_axis_name="s")
_sc_params = pltpu.CompilerParams(use_tc_tiling_on_sc=False)


@functools.partial(
    pl.kernel,
    out_type=jax.ShapeDtypeStruct((NC, N, DEGW), jnp.float32),
    mesh=_mesh,
    scratch_types=[
        pltpu.VMEM((NCH, CHUNK), jnp.int32),
        pltpu.VMEM((CHUNK, DEGW), jnp.float32),
        pltpu.VMEM_SHARED((N, DEGW), jnp.float32),
    ],
    compiler_params=_sc_params,
)
def _deg_kernel(row_hbm, ones_hbm, zeros_hbm, out_hbm, idx_v, ones_v, acc):
    c = lax.axis_index("c")
    s = lax.axis_index("s")
    wid = s * NC + c
    pltpu.sync_copy(row_hbm.at[wid], idx_v)
    pltpu.sync_copy(ones_hbm, ones_v)

    @pl.when(s == 0)
    def _():
        pltpu.sync_copy(zeros_hbm, acc)

    plsc.subcore_barrier()

    def body(j, carry):
        pltpu.sync_copy(ones_v, acc.at[idx_v.at[j]], add=True)
        return carry

    lax.fori_loop(0, NCH, body, 0)
    plsc.subcore_barrier()

    @pl.when(s == 0)
    def _():
        pltpu.sync_copy(acc, out_hbm.at[c])


@functools.partial(
    pl.kernel,
    out_type=jax.ShapeDtypeStruct((NC, N, DV), jnp.float32),
    mesh=_mesh,
    scratch_types=[
        pltpu.VMEM((NCH, CHUNK), jnp.int32),
        pltpu.VMEM((NCH, CHUNK), jnp.int32),
        pltpu.VMEM((CHUNK, DV), jnp.float32),
        pltpu.VMEM_SHARED((N, DV), jnp.float32),
    ],
    compiler_params=_sc_params,
)
def _spmm_kernel(v_hbm, col_hbm, row_hbm, zeros_hbm, out_hbm,
                 colv, rowv, buf, acc):
    c = lax.axis_index("c")
    s = lax.axis_index("s")
    wid = s * NC + c
    pltpu.sync_copy(col_hbm.at[wid], colv)
    pltpu.sync_copy(row_hbm.at[wid], rowv)

    @pl.when(s == 0)
    def _():
        pltpu.sync_copy(zeros_hbm, acc)

    plsc.subcore_barrier()

    def body(j, carry):
        pltpu.sync_copy(v_hbm.at[colv.at[j]], buf)
        pltpu.sync_copy(buf, acc.at[rowv.at[j]], add=True)
        return carry

    lax.fori_loop(0, NCH, body, 0)
    plsc.subcore_barrier()

    @pl.when(s == 0)
    def _():
        pltpu.sync_copy(acc, out_hbm.at[c])


def _prep_body(deg_ref, x_ref, m_ref, v_ref, dis_ref):
    deg = deg_ref[0, :, 0:1] + deg_ref[1, :, 0:1]
    dis = jnp.where(deg > 0.0, lax.rsqrt(jnp.maximum(deg, 1.0)), 0.0)
    g = dis * m_ref[...]
    xb = x_ref[...]
    xb = jnp.where(jnp.isnan(xb), 0.0, xb)
    v_ref[:, 0:D] = g * xb
    col16 = lax.broadcasted_iota(jnp.int32, (BN, DV - D), 1)
    tail = jnp.where(col16 == 0, g, jnp.where(col16 == 1, dis, 0.0))
    v_ref[:, D:DV] = tail
    dis_ref[...] = dis


def _final_body(vp_ref, dis_ref, w_ref, b_ref, o_ref):
    vs = vp_ref[0] + vp_ref[1]
    brow = vs[:, 0:D]
    tail = vs[:, D:DV]
    col16 = lax.broadcasted_iota(jnp.int32, (BN, DV - D), 1)
    cc = jnp.sum(jnp.where(col16 == 0, tail, 0.0), axis=1, keepdims=True)
    tt = jnp.sum(jnp.where(col16 == 1, tail, 0.0), axis=1, keepdims=True)
    dis = dis_ref[...]
    cz = cc == 0.0
    scale = jnp.where(cz, 0.0, dis * tt / jnp.where(cz, 1.0, cc))
    ratio = scale * brow
    acc = lax.dot_general(ratio, w_ref[...], (((1,), (1,)), ((), ())),
                          preferred_element_type=jnp.float32)
    o_ref[...] = acc + b_ref[...]


_prep = pl.pallas_call(
    _prep_body,
    grid=(N // BN,),
    in_specs=[
        pl.BlockSpec((NC, BN, DEGW), lambda i: (0, i, 0)),
        pl.BlockSpec((BN, D), lambda i: (i, 0)),
        pl.BlockSpec((BN, 1), lambda i: (i, 0)),
    ],
    out_specs=[
        pl.BlockSpec((BN, DV), lambda i: (i, 0)),
        pl.BlockSpec((BN, 1), lambda i: (i, 0)),
    ],
    out_shape=[
        jax.ShapeDtypeStruct((N, DV), jnp.float32),
        jax.ShapeDtypeStruct((N, 1), jnp.float32),
    ],
)

_final = pl.pallas_call(
    _final_body,
    grid=(N // BN,),
    in_specs=[
        pl.BlockSpec((NC, BN, DV), lambda i: (0, i, 0)),
        pl.BlockSpec((BN, 1), lambda i: (i, 0)),
        pl.BlockSpec((D, D), lambda i: (0, 0)),
        pl.BlockSpec((1, D), lambda i: (0, 0)),
    ],
    out_specs=pl.BlockSpec((BN, D), lambda i: (i, 0)),
    out_shape=jax.ShapeDtypeStruct((N, D), jnp.float32),
)


@jax.jit
def kernel(x, edge_index, train_mask, W, b):
    row = edge_index[0].reshape(NW, NCH, CHUNK)
    col = edge_index[1].reshape(NW, NCH, CHUNK)
    ones_deg = jnp.ones((CHUNK, DEGW), jnp.float32)
    zeros_deg = jnp.zeros((N, DEGW), jnp.float32)
    zeros_v = jnp.zeros((N, DV), jnp.float32)

    degp = _deg_kernel(row, ones_deg, zeros_deg)
    v, dis = _prep(degp, x, train_mask)
    vp = _spmm_kernel(v, col, row, zeros_v)
    out = _final(vp, dis, W, b.reshape(1, D))
    return out
